# Initial kernel scaffold; baseline (speedup 1.0000x reference)
import jax
import jax.numpy as jnp
from jax.experimental import pallas as pl


def kernel(feat, edge_index):
    raise NotImplementedError("write your pallas kernel here")


# speedup vs baseline: 5.3230x; 5.3230x over previous
import functools

import numpy as np
import jax
import jax.numpy as jnp
from jax import lax
from jax.experimental import pallas as pl
from jax.experimental.pallas import tpu as pltpu
from jax.experimental.pallas import tpu_sc as plsc

N_NODES = 10000
N_EDGES = 320000
D_FEAT = 128
K_HOPS = 10
ALPHA = 0.1

NC = 2
NS = 16
DH = D_FEAT // NC

CHUNK = 128
CHUNKS_PER_TILE = 157
E_PAD = NS * CHUNK * CHUNKS_PER_TILE
ROWS_PER_TILE = 640
N_PAD = ROWS_PER_TILE * NS
PAD_NODE = N_NODES
ROW_CHUNKS = ROWS_PER_TILE // CHUNK

_RSQRT_MAGIC = np.int32(0x5F3759DF)


def _vec_rsqrt(d):
    i = lax.bitcast_convert_type(d, jnp.int32)
    i = _RSQRT_MAGIC - lax.shift_right_logical(i, 1)
    y = lax.bitcast_convert_type(i, jnp.float32)
    for _ in range(3):
        y = y * (1.5 - 0.5 * d * y * y)
    return y


def _body(feat_hbm, src_hbm, dst_hbm, const_hbm,
          out_hbm, hs_hbm,
          accum_s, src_v, dst_v, rows_v, ebuf_a, ebuf_b, snorm_v, dnorm_v):
    c = lax.axis_index("c")
    s = lax.axis_index("s")
    base = s * ROWS_PER_TILE

    my_hs = hs_hbm.at[c]

    pltpu.sync_copy(src_hbm.at[s], src_v)
    pltpu.sync_copy(dst_hbm.at[s], dst_v)
    pltpu.sync_copy(const_hbm.at[0], rows_v)

    def zero_my_accum_rows():
        def zc(jc, _):
            pltpu.sync_copy(const_hbm.at[1],
                            accum_s.at[pl.ds(base + jc * CHUNK, CHUNK)])
            return 0
        lax.fori_loop(0, ROW_CHUNKS, zc, 0)

    def extract_norms(norm_ref):
        def nc_(jc, _):
            off = jc * CHUNK
            pltpu.sync_copy(accum_s.at[pl.ds(base + off, CHUNK)], ebuf_a)

            def nrow(r, _):
                d = jnp.maximum(ebuf_a[r, pl.ds(0, 16)], 1.0)
                norm_ref[off + r] = _vec_rsqrt(d)
                return 0
            lax.fori_loop(0, CHUNK, nrow, 0)
            return 0
        lax.fori_loop(0, ROW_CHUNKS, nc_, 0)

    zero_my_accum_rows()
    plsc.subcore_barrier()

    def deg_pass(idx_ref):
        def body(j, _):
            pltpu.sync_copy(rows_v, accum_s.at[idx_ref.at[j]], add=True)
            return 0
        lax.fori_loop(0, CHUNKS_PER_TILE, body, 0)

    deg_pass(src_v)
    plsc.subcore_barrier()
    extract_norms(snorm_v)
    zero_my_accum_rows()
    plsc.subcore_barrier()

    deg_pass(dst_v)
    plsc.subcore_barrier()
    extract_norms(dnorm_v)
    zero_my_accum_rows()

    def init_chunk(jc, _):
        off = jc * CHUNK
        pltpu.sync_copy(feat_hbm.at[c, pl.ds(base + off, CHUNK)], ebuf_b)

        def init_row(r, _):
            sn = snorm_v[off + r]
            for g in range(4):
                cs = pl.ds(g * 16, 16)
                ebuf_a[r, cs] = ebuf_b[r, cs] * sn
            return 0
        lax.fori_loop(0, CHUNK, init_row, 0)
        pltpu.sync_copy(ebuf_a, my_hs.at[pl.ds(base + off, CHUNK)])
        return 0
    lax.fori_loop(0, ROW_CHUNKS, init_chunk, 0)
    plsc.subcore_barrier()

    def hop(h, _):
        def edge_chunk(j, _):
            pltpu.sync_copy(my_hs.at[src_v.at[j]], rows_v)
            pltpu.sync_copy(rows_v, accum_s.at[dst_v.at[j]], add=True)
            return 0
        lax.fori_loop(0, CHUNKS_PER_TILE, edge_chunk, 0)
        plsc.subcore_barrier()

        def mix_chunk(jc, _):
            off = jc * CHUNK
            pltpu.sync_copy(accum_s.at[pl.ds(base + off, CHUNK)], ebuf_a)
            pltpu.sync_copy(feat_hbm.at[c, pl.ds(base + off, CHUNK)], ebuf_b)

            def mix_row(r, _):
                sn = snorm_v[off + r]
                dn = dnorm_v[off + r] * (1.0 - ALPHA)
                for g in range(4):
                    cs = pl.ds(g * 16, 16)
                    t = ebuf_a[r, cs] * dn + ebuf_b[r, cs] * ALPHA
                    ebuf_b[r, cs] = t
                    ebuf_a[r, cs] = t * sn
                return 0
            lax.fori_loop(0, CHUNK, mix_row, 0)
            pltpu.sync_copy(ebuf_b, out_hbm.at[c, pl.ds(base + off, CHUNK)])
            pltpu.sync_copy(ebuf_a, my_hs.at[pl.ds(base + off, CHUNK)])
            return 0
        lax.fori_loop(0, ROW_CHUNKS, mix_chunk, 0)

        zero_my_accum_rows()
        plsc.subcore_barrier()
        return 0

    lax.fori_loop(0, K_HOPS, hop, 0)


_sc_appnp = functools.partial(
    pl.kernel,
    out_type=(
        jax.ShapeDtypeStruct((NC, N_PAD, DH), jnp.float32),
        jax.ShapeDtypeStruct((NC, N_PAD, DH), jnp.float32),
    ),
    mesh=plsc.VectorSubcoreMesh(core_axis_name="c", subcore_axis_name="s"),
    compiler_params=pltpu.CompilerParams(use_tc_tiling_on_sc=False),
    scratch_types=[
        pltpu.VMEM_SHARED((N_PAD, DH), jnp.float32),
        pltpu.VMEM((CHUNKS_PER_TILE, CHUNK), jnp.int32),
        pltpu.VMEM((CHUNKS_PER_TILE, CHUNK), jnp.int32),
        pltpu.VMEM((CHUNK, DH), jnp.float32),
        pltpu.VMEM((CHUNK, DH), jnp.float32),
        pltpu.VMEM((CHUNK, DH), jnp.float32),
        pltpu.VMEM((ROWS_PER_TILE, 16), jnp.float32),
        pltpu.VMEM((ROWS_PER_TILE, 16), jnp.float32),
    ],
)(_body)


def kernel(feat, edge_index):
    feat = feat.astype(jnp.float32)
    ei = edge_index.astype(jnp.int32)

    pad = jnp.full((2, E_PAD - N_EDGES), PAD_NODE, jnp.int32)
    ei_pad = jnp.concatenate([ei, pad], axis=1)
    src_r = ei_pad[0].reshape(NS, CHUNKS_PER_TILE, CHUNK)
    dst_r = ei_pad[1].reshape(NS, CHUNKS_PER_TILE, CHUNK)

    fs = jnp.zeros((NC, N_PAD, DH), jnp.float32)
    fs = fs.at[0, :N_NODES].set(feat[:, :DH])
    fs = fs.at[1, :N_NODES].set(feat[:, DH:])

    consts = jnp.stack([jnp.ones((CHUNK, DH), jnp.float32),
                        jnp.zeros((CHUNK, DH), jnp.float32)])

    out, _hs = _sc_appnp(fs, src_r, dst_r, consts)
    return jnp.concatenate([out[0, :N_NODES], out[1, :N_NODES]], axis=1)


# speedup vs baseline: 6.6928x; 1.2574x over previous
import functools

import numpy as np
import jax
import jax.numpy as jnp
from jax import lax
from jax.experimental import pallas as pl
from jax.experimental.pallas import tpu as pltpu
from jax.experimental.pallas import tpu_sc as plsc

N_NODES = 10000
N_EDGES = 320000
D_FEAT = 128
K_HOPS = 10
ALPHA = 0.1

NC = 2
NS = 16
DH = D_FEAT // NC

CHUNK = 128
CHUNKS_PER_TILE = 157
E_PAD = NS * CHUNK * CHUNKS_PER_TILE
ROWS_PER_TILE = 640
N_PAD = ROWS_PER_TILE * NS
PAD_NODE = N_NODES
ROW_CHUNKS = ROWS_PER_TILE // CHUNK

_RSQRT_MAGIC = np.int32(0x5F3759DF)


def _vec_rsqrt(d):
    i = lax.bitcast_convert_type(d, jnp.int32)
    i = _RSQRT_MAGIC - lax.shift_right_logical(i, 1)
    y = lax.bitcast_convert_type(i, jnp.float32)
    for _ in range(3):
        y = y * (1.5 - 0.5 * d * y * y)
    return y


def _body(feat_hbm, src_hbm, dst_hbm, const_hbm,
          out_hbm, hs_hbm,
          accum_s, src_v, dst_v, rows2_v, snorm_v, dnorm_v, gsem):
    c = lax.axis_index("c")
    s = lax.axis_index("s")
    base = s * ROWS_PER_TILE

    my_hs = hs_hbm.at[c]

    ebuf_a = rows2_v.at[0]
    ebuf_b = rows2_v.at[1]

    pltpu.sync_copy(src_hbm.at[s], src_v)
    pltpu.sync_copy(dst_hbm.at[s], dst_v)
    pltpu.sync_copy(const_hbm.at[0], ebuf_a)

    def zero_my_accum_rows():
        def zc(jc, _):
            pltpu.sync_copy(const_hbm.at[1],
                            accum_s.at[pl.ds(base + jc * CHUNK, CHUNK)])
            return 0
        lax.fori_loop(0, ROW_CHUNKS, zc, 0)

    def extract_norms(norm_ref):
        def nc_(jc, _):
            off = jc * CHUNK
            pltpu.sync_copy(accum_s.at[pl.ds(base + off, CHUNK)], ebuf_b)

            def nrow(r, _):
                d = jnp.maximum(ebuf_b[r, pl.ds(0, 16)], 1.0)
                norm_ref[off + r] = _vec_rsqrt(d)
                return 0
            lax.fori_loop(0, CHUNK, nrow, 0)
            return 0
        lax.fori_loop(0, ROW_CHUNKS, nc_, 0)

    zero_my_accum_rows()
    plsc.subcore_barrier()

    def deg_pass(idx_ref):
        def body(j, _):
            pltpu.sync_copy(ebuf_a, accum_s.at[idx_ref.at[j]], add=True)
            return 0
        lax.fori_loop(0, CHUNKS_PER_TILE, body, 0)

    deg_pass(src_v)
    plsc.subcore_barrier()
    extract_norms(snorm_v)
    zero_my_accum_rows()
    plsc.subcore_barrier()

    deg_pass(dst_v)
    plsc.subcore_barrier()
    extract_norms(dnorm_v)
    zero_my_accum_rows()

    def init_chunk(jc, _):
        off = jc * CHUNK
        pltpu.sync_copy(feat_hbm.at[c, pl.ds(base + off, CHUNK)], ebuf_b)

        def init_row(r, _):
            sn = snorm_v[off + r]
            for g in range(4):
                cs = pl.ds(g * 16, 16)
                ebuf_a[r, cs] = ebuf_b[r, cs] * sn
            return 0
        lax.fori_loop(0, CHUNK, init_row, 0)
        pltpu.sync_copy(ebuf_a, my_hs.at[pl.ds(base + off, CHUNK)])
        return 0
    lax.fori_loop(0, ROW_CHUNKS, init_chunk, 0)
    plsc.subcore_barrier()

    def gather_start(j, p):
        pltpu.async_copy(my_hs.at[src_v.at[j]], rows2_v.at[p], gsem.at[p])

    def gather_wait(j, p):
        pltpu.make_async_copy(my_hs.at[src_v.at[j]], rows2_v.at[p],
                              gsem.at[p]).wait()

    def hop(h, _):
        gather_start(0, 0)

        def edge_chunk(j, _):
            p = lax.rem(j, 2)
            gather_wait(j, p)

            @pl.when(j + 1 < CHUNKS_PER_TILE)
            def _():
                gather_start(j + 1, 1 - p)

            pltpu.sync_copy(rows2_v.at[p], accum_s.at[dst_v.at[j]], add=True)
            return 0
        lax.fori_loop(0, CHUNKS_PER_TILE, edge_chunk, 0)
        plsc.subcore_barrier()

        def mix_chunk(jc, _):
            off = jc * CHUNK
            pltpu.sync_copy(accum_s.at[pl.ds(base + off, CHUNK)], ebuf_a)
            pltpu.sync_copy(feat_hbm.at[c, pl.ds(base + off, CHUNK)], ebuf_b)

            def mix_row(r, _):
                sn = snorm_v[off + r]
                dn = dnorm_v[off + r] * (1.0 - ALPHA)
                for g in range(4):
                    cs = pl.ds(g * 16, 16)
                    t = ebuf_a[r, cs] * dn + ebuf_b[r, cs] * ALPHA
                    ebuf_b[r, cs] = t
                    ebuf_a[r, cs] = t * sn
                return 0
            lax.fori_loop(0, CHUNK, mix_row, 0)
            pltpu.sync_copy(ebuf_b, out_hbm.at[c, pl.ds(base + off, CHUNK)])
            pltpu.sync_copy(ebuf_a, my_hs.at[pl.ds(base + off, CHUNK)])
            return 0
        lax.fori_loop(0, ROW_CHUNKS, mix_chunk, 0)

        zero_my_accum_rows()
        plsc.subcore_barrier()
        return 0

    lax.fori_loop(0, K_HOPS, hop, 0)


_sc_appnp = functools.partial(
    pl.kernel,
    out_type=(
        jax.ShapeDtypeStruct((NC, N_PAD, DH), jnp.float32),
        jax.ShapeDtypeStruct((NC, N_PAD, DH), jnp.float32),
    ),
    mesh=plsc.VectorSubcoreMesh(core_axis_name="c", subcore_axis_name="s"),
    compiler_params=pltpu.CompilerParams(use_tc_tiling_on_sc=False),
    scratch_types=[
        pltpu.VMEM_SHARED((N_PAD, DH), jnp.float32),
        pltpu.VMEM((CHUNKS_PER_TILE, CHUNK), jnp.int32),
        pltpu.VMEM((CHUNKS_PER_TILE, CHUNK), jnp.int32),
        pltpu.VMEM((2, CHUNK, DH), jnp.float32),
        pltpu.VMEM((ROWS_PER_TILE, 16), jnp.float32),
        pltpu.VMEM((ROWS_PER_TILE, 16), jnp.float32),
        pltpu.SemaphoreType.DMA((2,)),
    ],
)(_body)


def kernel(feat, edge_index):
    feat = feat.astype(jnp.float32)
    ei = edge_index.astype(jnp.int32)

    pad = jnp.full((2, E_PAD - N_EDGES), PAD_NODE, jnp.int32)
    ei_pad = jnp.concatenate([ei, pad], axis=1)
    src_r = ei_pad[0].reshape(NS, CHUNKS_PER_TILE, CHUNK)
    dst_r = ei_pad[1].reshape(NS, CHUNKS_PER_TILE, CHUNK)

    fs = jnp.zeros((NC, N_PAD, DH), jnp.float32)
    fs = fs.at[0, :N_NODES].set(feat[:, :DH])
    fs = fs.at[1, :N_NODES].set(feat[:, DH:])

    consts = jnp.stack([jnp.ones((CHUNK, DH), jnp.float32),
                        jnp.zeros((CHUNK, DH), jnp.float32)])

    out, _hs = _sc_appnp(fs, src_r, dst_r, consts)
    return jnp.concatenate([out[0, :N_NODES], out[1, :N_NODES]], axis=1)


# speedup vs baseline: 8.7217x; 1.3031x over previous
import functools

import numpy as np
import jax
import jax.numpy as jnp
from jax import lax
from jax.experimental import pallas as pl
from jax.experimental.pallas import tpu as pltpu
from jax.experimental.pallas import tpu_sc as plsc

N_NODES = 10000
N_EDGES = 320000
D_FEAT = 128
K_HOPS = 10
ALPHA = 0.1

NC = 2
NS = 16
DH = D_FEAT // NC

CHUNK = 128
CHUNKS_PER_TILE = 157
E_PAD = NS * CHUNK * CHUNKS_PER_TILE
ROWS_PER_TILE = 640
N_PAD = ROWS_PER_TILE * NS
PAD_NODE = N_NODES
ROW_CHUNKS = ROWS_PER_TILE // CHUNK

_RSQRT_MAGIC = np.int32(0x5F3759DF)


def _vec_rsqrt(d):
    i = lax.bitcast_convert_type(d, jnp.int32)
    i = _RSQRT_MAGIC - lax.shift_right_logical(i, 1)
    y = lax.bitcast_convert_type(i, jnp.float32)
    for _ in range(3):
        y = y * (1.5 - 0.5 * d * y * y)
    return y


def _body(feat_hbm, src_hbm, dst_hbm, const_hbm,
          out_hbm, hs_hbm,
          accum_s, src_v, dst_v, rows2_v, snorm_v, dnorm_v, gsem, ssem):
    c = lax.axis_index("c")
    s = lax.axis_index("s")
    base = s * ROWS_PER_TILE

    my_hs = hs_hbm.at[c]

    ebuf_a = rows2_v.at[0]
    ebuf_b = rows2_v.at[1]

    pltpu.sync_copy(src_hbm.at[s], src_v)
    pltpu.sync_copy(dst_hbm.at[s], dst_v)
    pltpu.sync_copy(const_hbm.at[0], ebuf_a)

    def zero_my_accum_rows():
        def zc(jc, _):
            pltpu.sync_copy(const_hbm.at[1],
                            accum_s.at[pl.ds(base + jc * CHUNK, CHUNK)])
            return 0
        lax.fori_loop(0, ROW_CHUNKS, zc, 0)

    def extract_norms(norm_ref):
        def nc_(jc, _):
            off = jc * CHUNK
            pltpu.sync_copy(accum_s.at[pl.ds(base + off, CHUNK)], ebuf_b)

            def nrow(r, _):
                d = jnp.maximum(ebuf_b[r, pl.ds(0, 16)], 1.0)
                norm_ref[off + r] = _vec_rsqrt(d)
                return 0
            lax.fori_loop(0, CHUNK, nrow, 0)
            return 0
        lax.fori_loop(0, ROW_CHUNKS, nc_, 0)

    zero_my_accum_rows()
    plsc.subcore_barrier()

    def deg_pass(idx_ref):
        def body(j, _):
            pltpu.sync_copy(ebuf_a, accum_s.at[idx_ref.at[j]], add=True)
            return 0
        lax.fori_loop(0, CHUNKS_PER_TILE, body, 0)

    deg_pass(src_v)
    plsc.subcore_barrier()
    extract_norms(snorm_v)
    zero_my_accum_rows()
    plsc.subcore_barrier()

    deg_pass(dst_v)
    plsc.subcore_barrier()
    extract_norms(dnorm_v)
    zero_my_accum_rows()

    def init_chunk(jc, _):
        off = jc * CHUNK
        pltpu.sync_copy(feat_hbm.at[c, pl.ds(base + off, CHUNK)], ebuf_b)

        def init_row(r, _):
            sn = snorm_v[off + r]
            for g in range(4):
                cs = pl.ds(g * 16, 16)
                ebuf_a[r, cs] = ebuf_b[r, cs] * sn
            return 0
        lax.fori_loop(0, CHUNK, init_row, 0)
        pltpu.sync_copy(ebuf_a, my_hs.at[pl.ds(base + off, CHUNK)])
        return 0
    lax.fori_loop(0, ROW_CHUNKS, init_chunk, 0)
    plsc.subcore_barrier()

    def gather_start(j, p):
        pltpu.async_copy(my_hs.at[src_v.at[j]], rows2_v.at[p], gsem.at[p])

    def gather_wait(j, p):
        pltpu.make_async_copy(my_hs.at[src_v.at[j]], rows2_v.at[p],
                              gsem.at[p]).wait()

    def scatter_start(j, p):
        pltpu.async_copy(rows2_v.at[p], accum_s.at[dst_v.at[j]], ssem.at[p],
                         add=True)

    def scatter_wait(j, p):
        pltpu.make_async_copy(rows2_v.at[p], accum_s.at[dst_v.at[j]],
                              ssem.at[p]).wait()

    def hop(h, _):
        gather_start(0, 0)
        gather_start(1, 1)

        def edge_chunk(j, _):
            p = lax.rem(j, 3)
            gather_wait(j, p)

            @pl.when(j >= 1)
            def _():
                scatter_wait(j - 1, lax.rem(j - 1, 3))

            @pl.when(j + 2 < CHUNKS_PER_TILE)
            def _():
                gather_start(j + 2, lax.rem(j + 2, 3))

            scatter_start(j, p)
            return 0
        lax.fori_loop(0, CHUNKS_PER_TILE, edge_chunk, 0)
        scatter_wait(CHUNKS_PER_TILE - 1, lax.rem(CHUNKS_PER_TILE - 1, 3))
        plsc.subcore_barrier()

        def mix_chunk(jc, _):
            off = jc * CHUNK
            pltpu.sync_copy(accum_s.at[pl.ds(base + off, CHUNK)], ebuf_a)
            pltpu.sync_copy(feat_hbm.at[c, pl.ds(base + off, CHUNK)], ebuf_b)

            def mix_row(r, _):
                sn = snorm_v[off + r]
                dn = dnorm_v[off + r] * (1.0 - ALPHA)
                for g in range(4):
                    cs = pl.ds(g * 16, 16)
                    t = ebuf_a[r, cs] * dn + ebuf_b[r, cs] * ALPHA
                    ebuf_b[r, cs] = t
                    ebuf_a[r, cs] = t * sn
                return 0
            lax.fori_loop(0, CHUNK, mix_row, 0)
            pltpu.sync_copy(ebuf_b, out_hbm.at[c, pl.ds(base + off, CHUNK)])
            pltpu.sync_copy(ebuf_a, my_hs.at[pl.ds(base + off, CHUNK)])
            return 0
        lax.fori_loop(0, ROW_CHUNKS, mix_chunk, 0)

        zero_my_accum_rows()
        plsc.subcore_barrier()
        return 0

    lax.fori_loop(0, K_HOPS, hop, 0)


_sc_appnp = functools.partial(
    pl.kernel,
    out_type=(
        jax.ShapeDtypeStruct((NC, N_PAD, DH), jnp.float32),
        jax.ShapeDtypeStruct((NC, N_PAD, DH), jnp.float32),
    ),
    mesh=plsc.VectorSubcoreMesh(core_axis_name="c", subcore_axis_name="s"),
    compiler_params=pltpu.CompilerParams(use_tc_tiling_on_sc=False),
    scratch_types=[
        pltpu.VMEM_SHARED((N_PAD, DH), jnp.float32),
        pltpu.VMEM((CHUNKS_PER_TILE, CHUNK), jnp.int32),
        pltpu.VMEM((CHUNKS_PER_TILE, CHUNK), jnp.int32),
        pltpu.VMEM((3, CHUNK, DH), jnp.float32),
        pltpu.VMEM((ROWS_PER_TILE, 16), jnp.float32),
        pltpu.VMEM((ROWS_PER_TILE, 16), jnp.float32),
        pltpu.SemaphoreType.DMA((3,)),
        pltpu.SemaphoreType.DMA((3,)),
    ],
)(_body)


def kernel(feat, edge_index):
    feat = feat.astype(jnp.float32)
    ei = edge_index.astype(jnp.int32)

    pad = jnp.full((2, E_PAD - N_EDGES), PAD_NODE, jnp.int32)
    ei_pad = jnp.concatenate([ei, pad], axis=1)
    src_r = ei_pad[0].reshape(NS, CHUNKS_PER_TILE, CHUNK)
    dst_r = ei_pad[1].reshape(NS, CHUNKS_PER_TILE, CHUNK)

    fs = jnp.zeros((NC, N_PAD, DH), jnp.float32)
    fs = fs.at[0, :N_NODES].set(feat[:, :DH])
    fs = fs.at[1, :N_NODES].set(feat[:, DH:])

    consts = jnp.stack([jnp.ones((CHUNK, DH), jnp.float32),
                        jnp.zeros((CHUNK, DH), jnp.float32)])

    out, _hs = _sc_appnp(fs, src_r, dst_r, consts)
    return jnp.concatenate([out[0, :N_NODES], out[1, :N_NODES]], axis=1)


# speedup vs baseline: 8.8194x; 1.0112x over previous
import functools

import numpy as np
import jax
import jax.numpy as jnp
from jax import lax
from jax.experimental import pallas as pl
from jax.experimental.pallas import tpu as pltpu
from jax.experimental.pallas import tpu_sc as plsc

N_NODES = 10000
N_EDGES = 320000
D_FEAT = 128
K_HOPS = 10
ALPHA = 0.1

NC = 2
NS = 16
DH = D_FEAT // NC

CHUNK = 128
CHUNKS_PER_TILE = 157
E_PAD = NS * CHUNK * CHUNKS_PER_TILE
ROWS_PER_TILE = 640
N_PAD = ROWS_PER_TILE * NS
PAD_NODE = N_NODES
ROW_CHUNKS = ROWS_PER_TILE // CHUNK

_RSQRT_MAGIC = np.int32(0x5F3759DF)


def _vec_rsqrt(d):
    i = lax.bitcast_convert_type(d, jnp.int32)
    i = _RSQRT_MAGIC - lax.shift_right_logical(i, 1)
    y = lax.bitcast_convert_type(i, jnp.float32)
    for _ in range(3):
        y = y * (1.5 - 0.5 * d * y * y)
    return y


def _body(feat_hbm, src_hbm, dst_hbm, const_hbm,
          out_hbm, hs_hbm,
          accum_s, src_v, dst_v, rows2_v, snorm_v, dnorm_v, gsem, ssem, zsem):
    c = lax.axis_index("c")
    s = lax.axis_index("s")
    base = s * ROWS_PER_TILE

    my_hs = hs_hbm.at[c]

    ebuf_a = rows2_v.at[0]
    ebuf_b = rows2_v.at[1]

    pltpu.sync_copy(src_hbm.at[s], src_v)
    pltpu.sync_copy(dst_hbm.at[s], dst_v)
    pltpu.sync_copy(const_hbm.at[0], ebuf_a)

    def zero_my_accum_rows():
        def zc(jc, _):
            pltpu.async_copy(const_hbm.at[1],
                             accum_s.at[pl.ds(base + jc * CHUNK, CHUNK)],
                             zsem)
            return 0
        lax.fori_loop(0, ROW_CHUNKS, zc, 0)

        def zw(jc, _):
            pltpu.make_async_copy(
                const_hbm.at[1],
                accum_s.at[pl.ds(base + jc * CHUNK, CHUNK)], zsem).wait()
            return 0
        lax.fori_loop(0, ROW_CHUNKS, zw, 0)

    def extract_norms(norm_ref):
        def nc_(jc, _):
            off = jc * CHUNK
            pltpu.sync_copy(accum_s.at[pl.ds(base + off, CHUNK)], ebuf_b)

            def nrow(r, _):
                d = jnp.maximum(ebuf_b[r, pl.ds(0, 16)], 1.0)
                norm_ref[off + r] = _vec_rsqrt(d)
                return 0
            lax.fori_loop(0, CHUNK, nrow, 0)
            return 0
        lax.fori_loop(0, ROW_CHUNKS, nc_, 0)

    zero_my_accum_rows()
    plsc.subcore_barrier()

    def deg_pass(idx_ref):
        def body(j, _):
            pltpu.async_copy(ebuf_a, accum_s.at[idx_ref.at[j]], zsem, add=True)
            return 0
        lax.fori_loop(0, CHUNKS_PER_TILE, body, 0)

        def drain(j, _):
            pltpu.make_async_copy(ebuf_a, accum_s.at[idx_ref.at[j]],
                                  zsem).wait()
            return 0
        lax.fori_loop(0, CHUNKS_PER_TILE, drain, 0)

    deg_pass(src_v)
    plsc.subcore_barrier()
    extract_norms(snorm_v)
    zero_my_accum_rows()
    plsc.subcore_barrier()

    deg_pass(dst_v)
    plsc.subcore_barrier()
    extract_norms(dnorm_v)
    zero_my_accum_rows()

    def init_chunk(jc, _):
        off = jc * CHUNK
        pltpu.sync_copy(feat_hbm.at[c, pl.ds(base + off, CHUNK)], ebuf_b)

        def init_row(r, _):
            sn = snorm_v[off + r]
            for g in range(4):
                cs = pl.ds(g * 16, 16)
                ebuf_a[r, cs] = ebuf_b[r, cs] * sn
            return 0
        lax.fori_loop(0, CHUNK, init_row, 0)
        pltpu.sync_copy(ebuf_a, my_hs.at[pl.ds(base + off, CHUNK)])
        return 0
    lax.fori_loop(0, ROW_CHUNKS, init_chunk, 0)
    plsc.subcore_barrier()

    def gather_start(j, p):
        pltpu.async_copy(my_hs.at[src_v.at[j]], rows2_v.at[p], gsem.at[p])

    def gather_wait(j, p):
        pltpu.make_async_copy(my_hs.at[src_v.at[j]], rows2_v.at[p],
                              gsem.at[p]).wait()

    def scatter_start(j, p):
        pltpu.async_copy(rows2_v.at[p], accum_s.at[dst_v.at[j]], ssem.at[p],
                         add=True)

    def scatter_wait(j, p):
        pltpu.make_async_copy(rows2_v.at[p], accum_s.at[dst_v.at[j]],
                              ssem.at[p]).wait()

    def hop(h, _):
        gather_start(0, 0)
        gather_start(1, 1)

        def edge_chunk(j, _):
            p = lax.rem(j, 3)
            gather_wait(j, p)

            @pl.when(j >= 1)
            def _():
                scatter_wait(j - 1, lax.rem(j - 1, 3))

            @pl.when(j + 2 < CHUNKS_PER_TILE)
            def _():
                gather_start(j + 2, lax.rem(j + 2, 3))

            scatter_start(j, p)
            return 0
        lax.fori_loop(0, CHUNKS_PER_TILE, edge_chunk, 0)
        scatter_wait(CHUNKS_PER_TILE - 1, lax.rem(CHUNKS_PER_TILE - 1, 3))
        plsc.subcore_barrier()

        def mix_chunk(jc, _):
            off = jc * CHUNK
            pltpu.sync_copy(accum_s.at[pl.ds(base + off, CHUNK)], ebuf_a)
            pltpu.sync_copy(feat_hbm.at[c, pl.ds(base + off, CHUNK)], ebuf_b)

            def mix_row(r, _):
                sn = snorm_v[off + r]
                dn = dnorm_v[off + r] * (1.0 - ALPHA)
                for g in range(4):
                    cs = pl.ds(g * 16, 16)
                    t = ebuf_a[r, cs] * dn + ebuf_b[r, cs] * ALPHA
                    ebuf_b[r, cs] = t
                    ebuf_a[r, cs] = t * sn
                return 0
            lax.fori_loop(0, CHUNK, mix_row, 0)

            @pl.when(h == K_HOPS - 1)
            def _():
                pltpu.sync_copy(ebuf_b,
                                out_hbm.at[c, pl.ds(base + off, CHUNK)])
            pltpu.sync_copy(ebuf_a, my_hs.at[pl.ds(base + off, CHUNK)])
            return 0
        lax.fori_loop(0, ROW_CHUNKS, mix_chunk, 0)

        zero_my_accum_rows()
        plsc.subcore_barrier()
        return 0

    lax.fori_loop(0, K_HOPS, hop, 0)


_sc_appnp = functools.partial(
    pl.kernel,
    out_type=(
        jax.ShapeDtypeStruct((NC, N_PAD, DH), jnp.float32),
        jax.ShapeDtypeStruct((NC, N_PAD, DH), jnp.float32),
    ),
    mesh=plsc.VectorSubcoreMesh(core_axis_name="c", subcore_axis_name="s"),
    compiler_params=pltpu.CompilerParams(use_tc_tiling_on_sc=False),
    scratch_types=[
        pltpu.VMEM_SHARED((N_PAD, DH), jnp.float32),
        pltpu.VMEM((CHUNKS_PER_TILE, CHUNK), jnp.int32),
        pltpu.VMEM((CHUNKS_PER_TILE, CHUNK), jnp.int32),
        pltpu.VMEM((3, CHUNK, DH), jnp.float32),
        pltpu.VMEM((ROWS_PER_TILE, 16), jnp.float32),
        pltpu.VMEM((ROWS_PER_TILE, 16), jnp.float32),
        pltpu.SemaphoreType.DMA((3,)),
        pltpu.SemaphoreType.DMA((3,)),
        pltpu.SemaphoreType.DMA,
    ],
)(_body)


def kernel(feat, edge_index):
    feat = feat.astype(jnp.float32)
    ei = edge_index.astype(jnp.int32)

    pad = jnp.full((2, E_PAD - N_EDGES), PAD_NODE, jnp.int32)
    ei_pad = jnp.concatenate([ei, pad], axis=1)
    src_r = ei_pad[0].reshape(NS, CHUNKS_PER_TILE, CHUNK)
    dst_r = ei_pad[1].reshape(NS, CHUNKS_PER_TILE, CHUNK)

    fs = jnp.zeros((NC, N_PAD, DH), jnp.float32)
    fs = fs.at[0, :N_NODES].set(feat[:, :DH])
    fs = fs.at[1, :N_NODES].set(feat[:, DH:])

    consts = jnp.stack([jnp.ones((CHUNK, DH), jnp.float32),
                        jnp.zeros((CHUNK, DH), jnp.float32)])

    out, _hs = _sc_appnp(fs, src_r, dst_r, consts)
    return jnp.concatenate([out[0, :N_NODES], out[1, :N_NODES]], axis=1)
